# baseline (speedup 1.0000x reference)
import functools

import jax
import jax.numpy as jnp
from jax.experimental import pallas as pl
from jax.experimental.pallas import tpu as pltpu

_GAM1 = 0.01
_GAM2 = 0.01


def _pge_tc_kernel(q_ref, p_ref, lab_ref, out_ref, p2_scr, *,
                   n_classes, n_total, np_):
    i = pl.program_id(0)

    @pl.when(i == 0)
    def _precompute_pivot_norms():
        ones8 = jnp.ones((8, q_ref.shape[1]), jnp.bfloat16)
        for k in range(np_):
            pk = p_ref[k]
            p2k = jax.lax.dot_general(
                ones8, pk * pk, (((1,), (1,)), ((), ())),
                preferred_element_type=jnp.float32)
            p2_scr[k] = p2k.astype(jnp.bfloat16)
    q = q_ref[...]
    bm = q.shape[0]
    q2 = jnp.sum(q * q, axis=1, keepdims=True)
    qm2 = (-2.0 * q).astype(jnp.bfloat16)

    big = jnp.float32(3.0e38)
    mn = jnp.full((bm, n_classes), big, jnp.bfloat16)
    mx = jnp.full((bm, n_classes), -big, jnp.bfloat16)
    for k in range(np_):
        qp = jax.lax.dot_general(
            qm2, p_ref[k], (((1,), (1,)), ((), ())),
            preferred_element_type=jnp.float32)
        t = qp.astype(jnp.bfloat16) + p2_scr[k, 0:1, :]
        mn = jnp.minimum(mn, t)
        mx = jnp.maximum(mx, t)

    mind = jnp.sqrt(jnp.maximum(mn.astype(jnp.float32) + q2, 1e-12))
    maxd = jnp.sqrt(jnp.maximum(mx.astype(jnp.float32) + q2, 1e-12))

    cls = jax.lax.broadcasted_iota(jnp.int32, (bm, n_classes), 1)
    own = lab_ref[...] == cls

    s_all_min = jnp.sum(mind)
    s_own_min = jnp.sum(jnp.where(own, mind, 0.0))
    s_own_max = jnp.sum(jnp.where(own, maxd, 0.0))

    part = (_GAM1 / n_total) * s_own_max \
        - (_GAM2 / (n_total * (n_classes - 1))) * (s_all_min - s_own_min)

    @pl.when(i == 0)
    def _init():
        out_ref[0, 0] = jnp.float32(0.0)

    out_ref[0, 0] += part


def kernel(queries, pivots, labels):
    B, d = queries.shape
    C, Np, _ = pivots.shape
    bm = 128

    p_t = jnp.transpose(pivots.astype(jnp.bfloat16), (1, 0, 2))
    lab = labels.astype(jnp.int32).reshape(B, 1)

    grid = (B // bm,)
    out = pl.pallas_call(
        functools.partial(_pge_tc_kernel, n_classes=C, n_total=B, np_=Np),
        grid=grid,
        in_specs=[
            pl.BlockSpec((bm, d), lambda i: (i, 0)),
            pl.BlockSpec((Np, C, d), lambda i: (0, 0, 0)),
            pl.BlockSpec((bm, 1), lambda i: (i, 0)),
        ],
        scratch_shapes=[pltpu.VMEM((Np, 8, C), jnp.bfloat16)],
        out_specs=pl.BlockSpec(memory_space=pltpu.SMEM),
        out_shape=jax.ShapeDtypeStruct((1, 1), jnp.float32),
        compiler_params=pltpu.CompilerParams(
            dimension_semantics=("arbitrary",)),
    )(queries, p_t, lab)
    return out[0, 0]


# speedup vs baseline: 1.0934x; 1.0934x over previous
import functools

import jax
import jax.numpy as jnp
from jax.experimental import pallas as pl
from jax.experimental.pallas import tpu as pltpu

_GAM1 = 0.01
_GAM2 = 0.01


def _pge_tc_kernel(q_ref, p_ref, lab_ref, out_ref, p2_scr, *,
                   n_classes, n_total, np_):
    i = pl.program_id(0)

    @pl.when(i == 0)
    def _precompute_pivot_norms():
        ones8 = jnp.ones((8, q_ref.shape[1]), jnp.bfloat16)
        for k in range(np_):
            pk = p_ref[k]
            p2k = jax.lax.dot_general(
                ones8, pk * pk, (((1,), (1,)), ((), ())),
                preferred_element_type=jnp.float32)
            p2_scr[k] = p2k.astype(jnp.bfloat16)
    q = q_ref[...]
    bm = q.shape[0]
    q2 = jnp.sum(q * q, axis=1, keepdims=True)
    qm2 = (-2.0 * q).astype(jnp.bfloat16)

    big = jnp.float32(3.0e38)
    s_all_min = jnp.float32(0.0)
    s_own_min = jnp.float32(0.0)
    s_own_max = jnp.float32(0.0)
    half = 256
    for c0 in range(0, n_classes, half):
        cw = min(half, n_classes - c0)
        mn = jnp.full((bm, cw), big, jnp.bfloat16)
        mx = jnp.full((bm, cw), -big, jnp.bfloat16)
        for k in range(np_):
            qp = jax.lax.dot_general(
                qm2, p_ref[k, c0:c0 + cw, :], (((1,), (1,)), ((), ())),
                preferred_element_type=jnp.float32)
            t = qp.astype(jnp.bfloat16) + p2_scr[k, 0:1, c0:c0 + cw]
            mn = jnp.minimum(mn, t)
            mx = jnp.maximum(mx, t)

        mind = jnp.sqrt(jnp.maximum(mn.astype(jnp.float32) + q2, 1e-12))
        maxd = jnp.sqrt(jnp.maximum(mx.astype(jnp.float32) + q2, 1e-12))

        cls = c0 + jax.lax.broadcasted_iota(jnp.int32, (bm, cw), 1)
        own = lab_ref[...] == cls

        s_all_min = s_all_min + jnp.sum(mind)
        s_own_min = s_own_min + jnp.sum(jnp.where(own, mind, 0.0))
        s_own_max = s_own_max + jnp.sum(jnp.where(own, maxd, 0.0))

    part = (_GAM1 / n_total) * s_own_max \
        - (_GAM2 / (n_total * (n_classes - 1))) * (s_all_min - s_own_min)

    @pl.when(i == 0)
    def _init():
        out_ref[0, 0] = jnp.float32(0.0)

    out_ref[0, 0] += part


def kernel(queries, pivots, labels):
    B, d = queries.shape
    C, Np, _ = pivots.shape
    bm = 256

    p_t = jnp.transpose(pivots.astype(jnp.bfloat16), (1, 0, 2))
    lab = labels.astype(jnp.int32).reshape(B, 1)

    grid = (B // bm,)
    out = pl.pallas_call(
        functools.partial(_pge_tc_kernel, n_classes=C, n_total=B, np_=Np),
        grid=grid,
        in_specs=[
            pl.BlockSpec((bm, d), lambda i: (i, 0)),
            pl.BlockSpec((Np, C, d), lambda i: (0, 0, 0)),
            pl.BlockSpec((bm, 1), lambda i: (i, 0)),
        ],
        scratch_shapes=[pltpu.VMEM((Np, 8, C), jnp.bfloat16)],
        out_specs=pl.BlockSpec(memory_space=pltpu.SMEM),
        out_shape=jax.ShapeDtypeStruct((1, 1), jnp.float32),
        compiler_params=pltpu.CompilerParams(
            dimension_semantics=("arbitrary",)),
    )(queries, p_t, lab)
    return out[0, 0]
